# baseline (speedup 1.0000x reference)
import functools

import jax
import jax.numpy as jnp
from jax import lax
from jax.experimental import pallas as pl
from jax.experimental.pallas import tpu as pltpu
from jax.experimental.pallas import tpu_sc as plsc

NC = 2
NS = 16
CHUNK = 128


def _sc_count(col1d, ones_hbm, zeros_hbm):
    NCHW = col1d.shape[0] // (NC * NS * CHUNK)
    Np = zeros_hbm.shape[0]
    CW = ones_hbm.shape[1]
    rows_per_s = Np // NS

    mesh = plsc.VectorSubcoreMesh(core_axis_name="c", subcore_axis_name="s", num_cores=NC, num_subcores=NS)

    def body(col_hbm, ones_h, zeros_h, out_hbm, coli_v, ones_v, acc_sh, sem):
        c = lax.axis_index("c")
        s = lax.axis_index("s")
        wid = c * NS + s
        base = wid * NCHW
        pltpu.sync_copy(ones_h, ones_v)
        pltpu.sync_copy(
            zeros_h.at[pl.ds(s * rows_per_s, rows_per_s)],
            acc_sh.at[pl.ds(s * rows_per_s, rows_per_s)],
        )
        plsc.subcore_barrier()

        nchw_t = NCHW + wid * 0

        @pl.loop(0, nchw_t, unroll=1)
        def _(k):
            pltpu.sync_copy(col_hbm.at[pl.ds((base + k) * CHUNK, CHUNK)], coli_v)
            pltpu.sync_copy(ones_v, acc_sh.at[coli_v], add=True)

        plsc.subcore_barrier()
        pltpu.sync_copy(
            acc_sh.at[pl.ds(s * rows_per_s, rows_per_s)],
            out_hbm.at[c, pl.ds(s * rows_per_s, rows_per_s)],
        )

    return pl.kernel(
        body,
        out_type=jax.ShapeDtypeStruct((NC, Np, CW), jnp.float32),
        mesh=mesh,
        scratch_types=[
            pltpu.VMEM((CHUNK,), jnp.int32),
            pltpu.VMEM((CHUNK, CW), jnp.float32),
            pltpu.VMEM_SHARED((Np, CW), jnp.float32),
            pltpu.SemaphoreType.DMA,
        ],
    )(col1d, ones_hbm, zeros_hbm)


def _sc_scatter(table, row1d, col1d, zeros_hbm):
    D = table.shape[1]
    NCHW = row1d.shape[0] // (NC * NS * CHUNK)
    Np = zeros_hbm.shape[0]
    rows_per_s = Np // NS

    mesh = plsc.VectorSubcoreMesh(core_axis_name="c", subcore_axis_name="s", num_cores=NC, num_subcores=NS)

    def body(table_hbm, row_hbm, col_hbm, zeros_h, out_hbm,
             rowi_v, coli_v, rows_v, acc_sh, gsem):
        c = lax.axis_index("c")
        s = lax.axis_index("s")
        wid = c * NS + s
        base = wid * NCHW
        pltpu.sync_copy(
            zeros_h.at[pl.ds(s * rows_per_s, rows_per_s)],
            acc_sh.at[pl.ds(s * rows_per_s, rows_per_s)],
        )
        plsc.subcore_barrier()

        nchw_t = NCHW + wid * 0

        @pl.loop(0, nchw_t, unroll=1)
        def _(k):
            ebase = (base + k) * CHUNK
            pltpu.sync_copy(row_hbm.at[pl.ds(ebase, CHUNK)], rowi_v)
            pltpu.sync_copy(col_hbm.at[pl.ds(ebase, CHUNK)], coli_v)
            pltpu.async_copy(table_hbm.at[rowi_v], rows_v, gsem).wait()
            pltpu.sync_copy(rows_v, acc_sh.at[coli_v], add=True)

        plsc.subcore_barrier()
        pltpu.sync_copy(
            acc_sh.at[pl.ds(s * rows_per_s, rows_per_s)],
            out_hbm.at[c, pl.ds(s * rows_per_s, rows_per_s)],
        )

    return pl.kernel(
        body,
        out_type=jax.ShapeDtypeStruct((NC, Np, D), jnp.float32),
        mesh=mesh,
        scratch_types=[
            pltpu.VMEM((CHUNK,), jnp.int32),
            pltpu.VMEM((CHUNK,), jnp.int32),
            pltpu.VMEM((CHUNK, D), jnp.float32),
            pltpu.VMEM_SHARED((Np, D), jnp.float32),
            pltpu.SemaphoreType.DMA,
        ],
    )(table, row1d, col1d, zeros_hbm)


_BLK = 2000


def _tc_stage_a(cnt2, x, W1):
    N, D = x.shape

    def body(cnt_ref, x_ref, w_ref, p1_ref, t1_ref, dinv_ref, invc_ref):
        cnt = cnt_ref[0][:, 0:1] + cnt_ref[1][:, 0:1]
        dinv = lax.rsqrt(cnt + 1.0)
        invc = 1.0 / jnp.maximum(cnt, 1.0)
        p1 = jnp.dot(x_ref[...], w_ref[...], preferred_element_type=jnp.float32)
        p1_ref[...] = p1
        t1_ref[...] = dinv * p1
        dinv_ref[...] = dinv
        invc_ref[...] = invc

    grid = N // _BLK
    return pl.pallas_call(
        body,
        grid=(grid,),
        in_specs=[
            pl.BlockSpec((NC, _BLK, 128), lambda i: (0, i, 0)),
            pl.BlockSpec((_BLK, D), lambda i: (i, 0)),
            pl.BlockSpec((D, D), lambda i: (0, 0)),
        ],
        out_specs=[
            pl.BlockSpec((_BLK, D), lambda i: (i, 0)),
            pl.BlockSpec((_BLK, D), lambda i: (i, 0)),
            pl.BlockSpec((_BLK, 1), lambda i: (i, 0)),
            pl.BlockSpec((_BLK, 1), lambda i: (i, 0)),
        ],
        out_shape=[
            jax.ShapeDtypeStruct((N, D), jnp.float32),
            jax.ShapeDtypeStruct((N, D), jnp.float32),
            jax.ShapeDtypeStruct((N, 1), jnp.float32),
            jax.ShapeDtypeStruct((N, 1), jnp.float32),
        ],
    )(cnt2, x, W1)


def _tc_stage_b(S, P, dinv, b, W):
    N, D = P.shape

    def body(s_ref, p_ref, dinv_ref, b_ref, w_ref, pn_ref, tn_ref):
        dv = dinv_ref[...]
        h = jax.nn.relu(dv * (s_ref[0] + s_ref[1] + dv * p_ref[...]) + b_ref[...])
        pn = jnp.dot(h, w_ref[...], preferred_element_type=jnp.float32)
        pn_ref[...] = pn
        tn_ref[...] = dv * pn

    grid = N // _BLK
    return pl.pallas_call(
        body,
        grid=(grid,),
        in_specs=[
            pl.BlockSpec((NC, _BLK, D), lambda i: (0, i, 0)),
            pl.BlockSpec((_BLK, D), lambda i: (i, 0)),
            pl.BlockSpec((_BLK, 1), lambda i: (i, 0)),
            pl.BlockSpec((1, D), lambda i: (0, 0)),
            pl.BlockSpec((D, D), lambda i: (0, 0)),
        ],
        out_specs=[
            pl.BlockSpec((_BLK, D), lambda i: (i, 0)),
            pl.BlockSpec((_BLK, D), lambda i: (i, 0)),
        ],
        out_shape=[
            jax.ShapeDtypeStruct((N, D), jnp.float32),
            jax.ShapeDtypeStruct((N, D), jnp.float32),
        ],
    )(S, P, dinv, b, W)


def _tc_stage_c(S, P, dinv, b):
    N, D = P.shape

    def body(s_ref, p_ref, dinv_ref, b_ref, h_ref):
        dv = dinv_ref[...]
        h_ref[...] = jax.nn.relu(
            dv * (s_ref[0] + s_ref[1] + dv * p_ref[...]) + b_ref[...]
        )

    grid = N // _BLK
    return pl.pallas_call(
        body,
        grid=(grid,),
        in_specs=[
            pl.BlockSpec((NC, _BLK, D), lambda i: (0, i, 0)),
            pl.BlockSpec((_BLK, D), lambda i: (i, 0)),
            pl.BlockSpec((_BLK, 1), lambda i: (i, 0)),
            pl.BlockSpec((1, D), lambda i: (0, 0)),
        ],
        out_specs=pl.BlockSpec((_BLK, D), lambda i: (i, 0)),
        out_shape=jax.ShapeDtypeStruct((N, D), jnp.float32),
    )(S, P, dinv, b)


def _tc_stage_d(AG, invc, H2, Wl, bl, Wr):
    N, D = H2.shape

    def body(ag_ref, invc_ref, h_ref, wl_ref, bl_ref, wr_ref, out_ref):
        mean = invc_ref[...] * (ag_ref[0] + ag_ref[1])
        out_ref[...] = (
            jnp.dot(mean, wl_ref[...], preferred_element_type=jnp.float32)
            + bl_ref[...]
            + jnp.dot(h_ref[...], wr_ref[...], preferred_element_type=jnp.float32)
        )

    grid = N // _BLK
    return pl.pallas_call(
        body,
        grid=(grid,),
        in_specs=[
            pl.BlockSpec((NC, _BLK, D), lambda i: (0, i, 0)),
            pl.BlockSpec((_BLK, 1), lambda i: (i, 0)),
            pl.BlockSpec((_BLK, D), lambda i: (i, 0)),
            pl.BlockSpec((D, D), lambda i: (0, 0)),
            pl.BlockSpec((1, D), lambda i: (0, 0)),
            pl.BlockSpec((D, D), lambda i: (0, 0)),
        ],
        out_specs=pl.BlockSpec((_BLK, D), lambda i: (i, 0)),
        out_shape=jax.ShapeDtypeStruct((N, D), jnp.float32),
    )(AG, invc, H2, Wl, bl, Wr)


def kernel(x, edge_index, W1, b1, W2, b2, Wl, bl, Wr):
    N, D = x.shape
    E = edge_index.shape[1]
    Npad = ((N + 8 * NS - 1) // (8 * NS)) * (8 * NS)
    nch = -(-E // (NC * NS * CHUNK))
    nch += nch % 2
    EPAD = NC * NS * nch * CHUNK
    row1d = jnp.concatenate(
        [edge_index[0], jnp.zeros((EPAD - E,), jnp.int32)])
    col1d = jnp.concatenate(
        [edge_index[1], jnp.full((EPAD - E,), Npad - 1, jnp.int32)])
    b1r = b1.reshape(1, D)
    b2r = b2.reshape(1, D)
    blr = bl.reshape(1, D)
    zeros_nd = jnp.zeros((Npad, D), jnp.float32)
    ones_cd = jnp.ones((CHUNK, D), jnp.float32)

    cnt2 = _sc_count(col1d, ones_cd, zeros_nd)
    P1, T1, dinv, invc = _tc_stage_a(cnt2, x, W1)
    S1 = _sc_scatter(T1, row1d, col1d, zeros_nd)
    P2, T2 = _tc_stage_b(S1, P1, dinv, b1r, W2)
    S2 = _sc_scatter(T2, row1d, col1d, zeros_nd)
    H2 = _tc_stage_c(S2, P2, dinv, b2r)
    AG = _sc_scatter(H2, row1d, col1d, zeros_nd)
    out = _tc_stage_d(AG, invc, H2, Wl, blr, Wr)
    return out


# speedup vs baseline: 1.0003x; 1.0003x over previous
import functools

import jax
import jax.numpy as jnp
from jax import lax
from jax.experimental import pallas as pl
from jax.experimental.pallas import tpu as pltpu
from jax.experimental.pallas import tpu_sc as plsc

NC = 2
NS = 16
CHUNK = 128


def _sc_count(col1d, ones_hbm, zeros_hbm):
    NCHW = col1d.shape[0] // (NC * NS * CHUNK)
    Np = zeros_hbm.shape[0]
    CW = ones_hbm.shape[1]
    rows_per_s = Np // NS

    mesh = plsc.VectorSubcoreMesh(core_axis_name="c", subcore_axis_name="s", num_cores=NC, num_subcores=NS)

    def body(col_hbm, ones_h, zeros_h, out_hbm, coli_v, ones_v, acc_sh, sem):
        c = lax.axis_index("c")
        s = lax.axis_index("s")
        wid = c * NS + s
        base = wid * NCHW
        pltpu.sync_copy(ones_h, ones_v)
        pltpu.sync_copy(
            zeros_h.at[pl.ds(s * rows_per_s, rows_per_s)],
            acc_sh.at[pl.ds(s * rows_per_s, rows_per_s)],
        )
        plsc.subcore_barrier()

        nchw_t = NCHW + wid * 0

        @pl.loop(0, nchw_t, unroll=1)
        def _(k):
            pltpu.sync_copy(col_hbm.at[pl.ds((base + k) * CHUNK, CHUNK)], coli_v)
            pltpu.sync_copy(ones_v, acc_sh.at[coli_v], add=True)

        plsc.subcore_barrier()
        pltpu.sync_copy(
            acc_sh.at[pl.ds(s * rows_per_s, rows_per_s)],
            out_hbm.at[c, pl.ds(s * rows_per_s, rows_per_s)],
        )

    return pl.kernel(
        body,
        out_type=jax.ShapeDtypeStruct((NC, Np, CW), jnp.float32),
        mesh=mesh,
        scratch_types=[
            pltpu.VMEM((CHUNK,), jnp.int32),
            pltpu.VMEM((CHUNK, CW), jnp.float32),
            pltpu.VMEM_SHARED((Np, CW), jnp.float32),
            pltpu.SemaphoreType.DMA,
        ],
    )(col1d, ones_hbm, zeros_hbm)


def _sc_scatter(table, row1d, col1d, zeros_hbm):
    D = table.shape[1]
    NCHW = row1d.shape[0] // (NC * NS * CHUNK)
    Np = zeros_hbm.shape[0]
    rows_per_s = Np // NS

    mesh = plsc.VectorSubcoreMesh(core_axis_name="c", subcore_axis_name="s", num_cores=NC, num_subcores=NS)

    def body(table_hbm, row_hbm, col_hbm, zeros_h, out_hbm,
             rowi_v, coli_v, rows_v, acc_sh, gsem):
        c = lax.axis_index("c")
        s = lax.axis_index("s")
        wid = c * NS + s
        base = wid * NCHW
        pltpu.sync_copy(
            zeros_h.at[pl.ds(s * rows_per_s, rows_per_s)],
            acc_sh.at[pl.ds(s * rows_per_s, rows_per_s)],
        )
        plsc.subcore_barrier()

        nchw_t = NCHW + wid * 0

        @pl.loop(0, nchw_t, unroll=1)
        def _(k):
            ebase = (base + k) * CHUNK
            pltpu.sync_copy(row_hbm.at[pl.ds(ebase, CHUNK)], rowi_v)
            pltpu.sync_copy(col_hbm.at[pl.ds(ebase, CHUNK)], coli_v)
            pltpu.async_copy(table_hbm.at[rowi_v], rows_v, gsem).wait()
            pltpu.sync_copy(rows_v, acc_sh.at[coli_v], add=True)

        plsc.subcore_barrier()
        pltpu.sync_copy(
            acc_sh.at[pl.ds(s * rows_per_s, rows_per_s)],
            out_hbm.at[c, pl.ds(s * rows_per_s, rows_per_s)],
        )

    return pl.kernel(
        body,
        out_type=jax.ShapeDtypeStruct((NC, Np, D), jnp.float32),
        mesh=mesh,
        scratch_types=[
            pltpu.VMEM((CHUNK,), jnp.int32),
            pltpu.VMEM((CHUNK,), jnp.int32),
            pltpu.VMEM((CHUNK, D), jnp.float32),
            pltpu.VMEM_SHARED((Np, D), jnp.float32),
            pltpu.SemaphoreType.DMA,
        ],
    )(table, row1d, col1d, zeros_hbm)


_BLK = 2000


def _tc_stage_a(cnt2, x, W1):
    N, D = x.shape

    def body(cnt_ref, x_ref, w_ref, p1_ref, t1_ref, dinv_ref, invc_ref):
        cnt = cnt_ref[0][:, 0:1] + cnt_ref[1][:, 0:1]
        dinv = lax.rsqrt(cnt + 1.0)
        invc = 1.0 / jnp.maximum(cnt, 1.0)
        p1 = jnp.dot(x_ref[...], w_ref[...], preferred_element_type=jnp.float32)
        p1_ref[...] = p1
        t1_ref[...] = dinv * p1
        dinv_ref[...] = dinv
        invc_ref[...] = invc

    grid = N // _BLK
    return pl.pallas_call(
        body,
        grid=(grid,),
        in_specs=[
            pl.BlockSpec((NC, _BLK, 128), lambda i: (0, i, 0)),
            pl.BlockSpec((_BLK, D), lambda i: (i, 0)),
            pl.BlockSpec((D, D), lambda i: (0, 0)),
        ],
        out_specs=[
            pl.BlockSpec((_BLK, D), lambda i: (i, 0)),
            pl.BlockSpec((_BLK, D), lambda i: (i, 0)),
            pl.BlockSpec((_BLK, 1), lambda i: (i, 0)),
            pl.BlockSpec((_BLK, 1), lambda i: (i, 0)),
        ],
        out_shape=[
            jax.ShapeDtypeStruct((N, D), jnp.float32),
            jax.ShapeDtypeStruct((N, D), jnp.float32),
            jax.ShapeDtypeStruct((N, 1), jnp.float32),
            jax.ShapeDtypeStruct((N, 1), jnp.float32),
        ],
    )(cnt2, x, W1)


def _tc_stage_b(S, P, dinv, b, W):
    N, D = P.shape

    def body(s_ref, p_ref, dinv_ref, b_ref, w_ref, pn_ref, tn_ref):
        dv = dinv_ref[...]
        h = jax.nn.relu(dv * (s_ref[0] + s_ref[1] + dv * p_ref[...]) + b_ref[...])
        pn = jnp.dot(h, w_ref[...], preferred_element_type=jnp.float32)
        pn_ref[...] = pn
        tn_ref[...] = dv * pn

    grid = N // _BLK
    return pl.pallas_call(
        body,
        grid=(grid,),
        in_specs=[
            pl.BlockSpec((NC, _BLK, D), lambda i: (0, i, 0)),
            pl.BlockSpec((_BLK, D), lambda i: (i, 0)),
            pl.BlockSpec((_BLK, 1), lambda i: (i, 0)),
            pl.BlockSpec((1, D), lambda i: (0, 0)),
            pl.BlockSpec((D, D), lambda i: (0, 0)),
        ],
        out_specs=[
            pl.BlockSpec((_BLK, D), lambda i: (i, 0)),
            pl.BlockSpec((_BLK, D), lambda i: (i, 0)),
        ],
        out_shape=[
            jax.ShapeDtypeStruct((N, D), jnp.float32),
            jax.ShapeDtypeStruct((N, D), jnp.float32),
        ],
    )(S, P, dinv, b, W)


def _tc_stage_c(S, P, dinv, b):
    N, D = P.shape

    def body(s_ref, p_ref, dinv_ref, b_ref, h_ref):
        dv = dinv_ref[...]
        h_ref[...] = jax.nn.relu(
            dv * (s_ref[0] + s_ref[1] + dv * p_ref[...]) + b_ref[...]
        )

    grid = N // _BLK
    return pl.pallas_call(
        body,
        grid=(grid,),
        in_specs=[
            pl.BlockSpec((NC, _BLK, D), lambda i: (0, i, 0)),
            pl.BlockSpec((_BLK, D), lambda i: (i, 0)),
            pl.BlockSpec((_BLK, 1), lambda i: (i, 0)),
            pl.BlockSpec((1, D), lambda i: (0, 0)),
        ],
        out_specs=pl.BlockSpec((_BLK, D), lambda i: (i, 0)),
        out_shape=jax.ShapeDtypeStruct((N, D), jnp.float32),
    )(S, P, dinv, b)


def _tc_stage_d(AG, invc, H2, Wl, bl, Wr):
    N, D = H2.shape

    def body(ag_ref, invc_ref, h_ref, wl_ref, bl_ref, wr_ref, out_ref):
        mean = invc_ref[...] * (ag_ref[0] + ag_ref[1])
        out_ref[...] = (
            jnp.dot(mean, wl_ref[...], preferred_element_type=jnp.float32)
            + bl_ref[...]
            + jnp.dot(h_ref[...], wr_ref[...], preferred_element_type=jnp.float32)
        )

    grid = N // _BLK
    return pl.pallas_call(
        body,
        grid=(grid,),
        in_specs=[
            pl.BlockSpec((NC, _BLK, D), lambda i: (0, i, 0)),
            pl.BlockSpec((_BLK, 1), lambda i: (i, 0)),
            pl.BlockSpec((_BLK, D), lambda i: (i, 0)),
            pl.BlockSpec((D, D), lambda i: (0, 0)),
            pl.BlockSpec((1, D), lambda i: (0, 0)),
            pl.BlockSpec((D, D), lambda i: (0, 0)),
        ],
        out_specs=pl.BlockSpec((_BLK, D), lambda i: (i, 0)),
        out_shape=jax.ShapeDtypeStruct((N, D), jnp.float32),
    )(AG, invc, H2, Wl, bl, Wr)


def kernel(x, edge_index, W1, b1, W2, b2, Wl, bl, Wr):
    N, D = x.shape
    E = edge_index.shape[1]
    Npad = ((N + 8 * NS - 1) // (8 * NS)) * (8 * NS)
    nch = -(-E // (NC * NS * CHUNK))
    nch += nch % 2
    EPAD = NC * NS * nch * CHUNK
    row1d = jnp.concatenate(
        [edge_index[0], jnp.zeros((EPAD - E,), jnp.int32)])
    pad_cols = N + jax.lax.rem(
        jnp.arange(EPAD - E, dtype=jnp.int32), jnp.int32(Npad - N))
    col1d = jnp.concatenate([edge_index[1], pad_cols])
    b1r = b1.reshape(1, D)
    b2r = b2.reshape(1, D)
    blr = bl.reshape(1, D)
    zeros_nd = jnp.zeros((Npad, D), jnp.float32)
    ones_cd = jnp.ones((CHUNK, D), jnp.float32)

    cnt2 = _sc_count(col1d, ones_cd, zeros_nd)
    P1, T1, dinv, invc = _tc_stage_a(cnt2, x, W1)
    S1 = _sc_scatter(T1, row1d, col1d, zeros_nd)
    P2, T2 = _tc_stage_b(S1, P1, dinv, b1r, W2)
    S2 = _sc_scatter(T2, row1d, col1d, zeros_nd)
    H2 = _tc_stage_c(S2, P2, dinv, b2r)
    AG = _sc_scatter(H2, row1d, col1d, zeros_nd)
    out = _tc_stage_d(AG, invc, H2, Wl, blr, Wr)
    return out


# speedup vs baseline: 2.0277x; 2.0270x over previous
import functools

import jax
import jax.numpy as jnp
from jax import lax
from jax.experimental import pallas as pl
from jax.experimental.pallas import tpu as pltpu
from jax.experimental.pallas import tpu_sc as plsc

NC = 2
NS = 16
CHUNK = 128


def _sc_count(col, ones_hbm, zeros_hbm):
    (E,) = col.shape
    Np = zeros_hbm.shape[0]
    CW = ones_hbm.shape[1]
    nchunks = E // CHUNK
    base, rem = divmod(nchunks, NC * NS)
    rows_per_s = Np // NS

    mesh = plsc.VectorSubcoreMesh(core_axis_name="c", subcore_axis_name="s", num_cores=NC, num_subcores=NS)

    def body(col_hbm, ones_h, zeros_h, out_hbm, coli_v, ones_v, acc_sh, sem):
        c = lax.axis_index("c")
        s = lax.axis_index("s")
        wid = c * NS + s
        pltpu.sync_copy(ones_h, ones_v)
        pltpu.sync_copy(
            zeros_h.at[pl.ds(s * rows_per_s, rows_per_s)],
            acc_sh.at[pl.ds(s * rows_per_s, rows_per_s)],
        )
        plsc.subcore_barrier()

        start = wid * base + jnp.minimum(wid, rem)
        count = base + jnp.where(wid < rem, 1, 0)

        @pl.loop(0, count)
        def _(j):
            ebase = (start + j) * CHUNK
            pltpu.sync_copy(col_hbm.at[pl.ds(ebase, CHUNK)], coli_v)
            pltpu.sync_copy(ones_v, acc_sh.at[coli_v], add=True)

        plsc.subcore_barrier()
        pltpu.sync_copy(
            acc_sh.at[pl.ds(s * rows_per_s, rows_per_s)],
            out_hbm.at[c, pl.ds(s * rows_per_s, rows_per_s)],
        )

    return pl.kernel(
        body,
        out_type=jax.ShapeDtypeStruct((NC, Np, CW), jnp.float32),
        mesh=mesh,
        scratch_types=[
            pltpu.VMEM((CHUNK,), jnp.int32),
            pltpu.VMEM((CHUNK, CW), jnp.float32),
            pltpu.VMEM_SHARED((Np, CW), jnp.float32),
            pltpu.SemaphoreType.DMA,
        ],
    )(col, ones_hbm, zeros_hbm)


def _sc_scatter(table, row, col, zeros_hbm):
    D = table.shape[1]
    (E,) = row.shape
    Np = zeros_hbm.shape[0]
    nchunks = E // CHUNK
    base, rem = divmod(nchunks, NC * NS)
    rows_per_s = Np // NS

    mesh = plsc.VectorSubcoreMesh(core_axis_name="c", subcore_axis_name="s", num_cores=NC, num_subcores=NS)

    def body(table_hbm, row_hbm, col_hbm, zeros_h, out_hbm,
             rowi_v, coli_v, rows_v, acc_sh, sem):
        c = lax.axis_index("c")
        s = lax.axis_index("s")
        wid = c * NS + s
        pltpu.sync_copy(
            zeros_h.at[pl.ds(s * rows_per_s, rows_per_s)],
            acc_sh.at[pl.ds(s * rows_per_s, rows_per_s)],
        )
        plsc.subcore_barrier()

        start = wid * base + jnp.minimum(wid, rem)
        count = base + jnp.where(wid < rem, 1, 0)

        @pl.loop(0, count)
        def _(j):
            ebase = (start + j) * CHUNK
            pltpu.sync_copy(row_hbm.at[pl.ds(ebase, CHUNK)], rowi_v)
            pltpu.sync_copy(col_hbm.at[pl.ds(ebase, CHUNK)], coli_v)
            pltpu.async_copy(table_hbm.at[rowi_v], rows_v, sem).wait()
            pltpu.sync_copy(rows_v, acc_sh.at[coli_v], add=True)

        plsc.subcore_barrier()
        pltpu.sync_copy(
            acc_sh.at[pl.ds(s * rows_per_s, rows_per_s)],
            out_hbm.at[c, pl.ds(s * rows_per_s, rows_per_s)],
        )

    return pl.kernel(
        body,
        out_type=jax.ShapeDtypeStruct((NC, Np, D), jnp.float32),
        mesh=mesh,
        scratch_types=[
            pltpu.VMEM((CHUNK,), jnp.int32),
            pltpu.VMEM((CHUNK,), jnp.int32),
            pltpu.VMEM((CHUNK, D), jnp.float32),
            pltpu.VMEM_SHARED((Np, D), jnp.float32),
            pltpu.SemaphoreType.DMA,
        ],
    )(table, row, col, zeros_hbm)


_BLK = 2000


def _tc_stage_a(cnt2, x, W1):
    N, D = x.shape

    def body(cnt_ref, x_ref, w_ref, p1_ref, t1_ref, dinv_ref, invc_ref):
        cnt = cnt_ref[0][:, 0:1] + cnt_ref[1][:, 0:1]
        dinv = lax.rsqrt(cnt + 1.0)
        invc = 1.0 / jnp.maximum(cnt, 1.0)
        p1 = jnp.dot(x_ref[...], w_ref[...], preferred_element_type=jnp.float32)
        p1_ref[...] = p1
        t1_ref[...] = dinv * p1
        dinv_ref[...] = dinv
        invc_ref[...] = invc

    grid = N // _BLK
    return pl.pallas_call(
        body,
        grid=(grid,),
        in_specs=[
            pl.BlockSpec((NC, _BLK, 128), lambda i: (0, i, 0)),
            pl.BlockSpec((_BLK, D), lambda i: (i, 0)),
            pl.BlockSpec((D, D), lambda i: (0, 0)),
        ],
        out_specs=[
            pl.BlockSpec((_BLK, D), lambda i: (i, 0)),
            pl.BlockSpec((_BLK, D), lambda i: (i, 0)),
            pl.BlockSpec((_BLK, 1), lambda i: (i, 0)),
            pl.BlockSpec((_BLK, 1), lambda i: (i, 0)),
        ],
        out_shape=[
            jax.ShapeDtypeStruct((N, D), jnp.float32),
            jax.ShapeDtypeStruct((N, D), jnp.float32),
            jax.ShapeDtypeStruct((N, 1), jnp.float32),
            jax.ShapeDtypeStruct((N, 1), jnp.float32),
        ],
    )(cnt2, x, W1)


def _tc_stage_b(S, P, dinv, b, W):
    N, D = P.shape

    def body(s_ref, p_ref, dinv_ref, b_ref, w_ref, pn_ref, tn_ref):
        dv = dinv_ref[...]
        h = jax.nn.relu(dv * (s_ref[0] + s_ref[1] + dv * p_ref[...]) + b_ref[...])
        pn = jnp.dot(h, w_ref[...], preferred_element_type=jnp.float32)
        pn_ref[...] = pn
        tn_ref[...] = dv * pn

    grid = N // _BLK
    return pl.pallas_call(
        body,
        grid=(grid,),
        in_specs=[
            pl.BlockSpec((NC, _BLK, D), lambda i: (0, i, 0)),
            pl.BlockSpec((_BLK, D), lambda i: (i, 0)),
            pl.BlockSpec((_BLK, 1), lambda i: (i, 0)),
            pl.BlockSpec((1, D), lambda i: (0, 0)),
            pl.BlockSpec((D, D), lambda i: (0, 0)),
        ],
        out_specs=[
            pl.BlockSpec((_BLK, D), lambda i: (i, 0)),
            pl.BlockSpec((_BLK, D), lambda i: (i, 0)),
        ],
        out_shape=[
            jax.ShapeDtypeStruct((N, D), jnp.float32),
            jax.ShapeDtypeStruct((N, D), jnp.float32),
        ],
    )(S, P, dinv, b, W)


def _tc_stage_c(S, P, dinv, b):
    N, D = P.shape

    def body(s_ref, p_ref, dinv_ref, b_ref, h_ref):
        dv = dinv_ref[...]
        h_ref[...] = jax.nn.relu(
            dv * (s_ref[0] + s_ref[1] + dv * p_ref[...]) + b_ref[...]
        )

    grid = N // _BLK
    return pl.pallas_call(
        body,
        grid=(grid,),
        in_specs=[
            pl.BlockSpec((NC, _BLK, D), lambda i: (0, i, 0)),
            pl.BlockSpec((_BLK, D), lambda i: (i, 0)),
            pl.BlockSpec((_BLK, 1), lambda i: (i, 0)),
            pl.BlockSpec((1, D), lambda i: (0, 0)),
        ],
        out_specs=pl.BlockSpec((_BLK, D), lambda i: (i, 0)),
        out_shape=jax.ShapeDtypeStruct((N, D), jnp.float32),
    )(S, P, dinv, b)


def _tc_stage_d(AG, invc, H2, Wl, bl, Wr):
    N, D = H2.shape

    def body(ag_ref, invc_ref, h_ref, wl_ref, bl_ref, wr_ref, out_ref):
        mean = invc_ref[...] * (ag_ref[0] + ag_ref[1])
        out_ref[...] = (
            jnp.dot(mean, wl_ref[...], preferred_element_type=jnp.float32)
            + bl_ref[...]
            + jnp.dot(h_ref[...], wr_ref[...], preferred_element_type=jnp.float32)
        )

    grid = N // _BLK
    return pl.pallas_call(
        body,
        grid=(grid,),
        in_specs=[
            pl.BlockSpec((NC, _BLK, D), lambda i: (0, i, 0)),
            pl.BlockSpec((_BLK, 1), lambda i: (i, 0)),
            pl.BlockSpec((_BLK, D), lambda i: (i, 0)),
            pl.BlockSpec((D, D), lambda i: (0, 0)),
            pl.BlockSpec((1, D), lambda i: (0, 0)),
            pl.BlockSpec((D, D), lambda i: (0, 0)),
        ],
        out_specs=pl.BlockSpec((_BLK, D), lambda i: (i, 0)),
        out_shape=jax.ShapeDtypeStruct((N, D), jnp.float32),
    )(AG, invc, H2, Wl, bl, Wr)


def kernel(x, edge_index, W1, b1, W2, b2, Wl, bl, Wr):
    N, D = x.shape
    row = edge_index[0]
    col = edge_index[1]
    b1r = b1.reshape(1, D)
    b2r = b2.reshape(1, D)
    blr = bl.reshape(1, D)
    Npad = ((N + 8 * NS - 1) // (8 * NS)) * (8 * NS)
    zeros_nd = jnp.zeros((Npad, D), jnp.float32)
    ones_cd = jnp.ones((CHUNK, D), jnp.float32)

    cnt2 = _sc_count(col, ones_cd, zeros_nd)
    P1, T1, dinv, invc = _tc_stage_a(cnt2, x, W1)
    S1 = _sc_scatter(T1, row, col, zeros_nd)
    P2, T2 = _tc_stage_b(S1, P1, dinv, b1r, W2)
    S2 = _sc_scatter(T2, row, col, zeros_nd)
    H2 = _tc_stage_c(S2, P2, dinv, b2r)
    AG = _sc_scatter(H2, row, col, zeros_nd)
    out = _tc_stage_d(AG, invc, H2, Wl, blr, Wr)
    return out
